# baseline (speedup 1.0000x reference)
import functools

import jax
import jax.numpy as jnp
from jax import lax
from jax.experimental import pallas as pl
from jax.experimental.pallas import tpu as pltpu
from jax.experimental.pallas import tpu_sc as plsc

DQ = 128
DM = 512
NC = 2
NS = 16
NW = NC * NS


@functools.lru_cache(maxsize=None)
def _make_sc_gather(bpad: int):
    bpw = bpad // NW
    mesh = plsc.VectorSubcoreMesh(core_axis_name="c", subcore_axis_name="s")

    @functools.partial(
        pl.kernel,
        mesh=mesh,
        out_type=jax.ShapeDtypeStruct((bpad, DQ), jnp.float32),
        scratch_types=[
            pltpu.VMEM((bpw,), jnp.int32),
            pltpu.VMEM((bpw, DQ), jnp.float32),
            pltpu.SemaphoreType.DMA,
        ],
    )
    def sc_gather(table_hbm, idx_hbm, out_hbm, idx_v, rows_v, sem):
        wid = lax.axis_index("s") * NC + lax.axis_index("c")
        base = wid * bpw
        pltpu.sync_copy(idx_hbm.at[pl.ds(base, bpw)], idx_v)
        pltpu.async_copy(table_hbm.at[idx_v], rows_v, sem).wait()
        pltpu.sync_copy(rows_v, out_hbm.at[pl.ds(base, bpw)])

    return sc_gather


def _tc_combine(g_ref, p_ref, sw_ref, sb_ref, w_ref, b_ref, o_ref, *, h, w):
    wc = w_ref[0:DQ, :]
    wp = w_ref[DQ:2 * DQ, :]
    ws = w_ref[2 * DQ:3 * DQ, :]
    size_e = float(h) * sw_ref[0:1, :] + float(w) * sw_ref[1:2, :] + sb_ref[0:1, :]
    const = jnp.dot(size_e, ws, preferred_element_type=jnp.float32) + b_ref[0:1, :]
    acc = jnp.dot(g_ref[...], wc, preferred_element_type=jnp.float32)
    acc = acc + jnp.dot(p_ref[...], wp, preferred_element_type=jnp.float32)
    o_ref[...] = acc + const


def kernel(grid, color_table, pos_emb, size_W, size_b, combine_W, combine_b):
    h, w = grid.shape
    n = h * w
    bpad = -(-n // (8 * NW)) * (8 * NW)

    idx = jnp.pad(grid.reshape(n).astype(jnp.int32), (0, bpad - n))
    pos = jnp.pad(pos_emb[:h, :w].reshape(n, DQ), ((0, bpad - n), (0, 0)))

    gathered = _make_sc_gather(bpad)(color_table, idx)

    out = pl.pallas_call(
        functools.partial(_tc_combine, h=h, w=w),
        out_shape=jax.ShapeDtypeStruct((bpad, DM), jnp.float32),
    )(
        gathered,
        pos,
        size_W,
        size_b.reshape(1, DQ),
        combine_W,
        combine_b.reshape(1, DM),
    )
    return out[:n].reshape(h, w, DM)
